# baseline (speedup 1.0000x reference)
import functools

import jax
import jax.numpy as jnp
from jax import lax
from jax.experimental import pallas as pl
from jax.experimental.pallas import tpu as pltpu
from jax.experimental.pallas import tpu_sc as plsc

_L = 16


@functools.cache
def _build(batch: int, n_fields: int, vocab: int):
    info = plsc.get_sparse_core_info()
    nc, ns = info.num_cores, info.num_subcores
    nw = nc * ns
    total = batch * n_fields
    assert total % (nw * _L) == 0
    per_w = total // nw
    n_steps = per_w // _L

    mesh = plsc.VectorSubcoreMesh(
        core_axis_name="c", subcore_axis_name="s", num_cores=nc,
        num_subcores=ns)

    @functools.partial(
        pl.kernel,
        out_type=jax.ShapeDtypeStruct((total,), jnp.float32),
        mesh=mesh,
        scratch_types=[
            pltpu.VMEM((per_w,), jnp.float32),
            pltpu.VMEM((per_w,), jnp.int32),
            pltpu.VMEM((per_w,), jnp.float32),
            pltpu.SemaphoreType.DMA,
        ],
    )
    def gather_kernel(table_hbm, xc_hbm, out_hbm, x_v, idx_v, out_v, sem):
        wid = lax.axis_index("s") * nc + lax.axis_index("c")
        base = wid * per_w
        pltpu.sync_copy(xc_hbm.at[pl.ds(base, per_w)], x_v)

        lane = lax.iota(jnp.int32, _L)
        shift = batch.bit_length() - 1
        assert batch == 1 << shift

        def body(j, _):
            pos = base + j * _L + lane
            col = lax.shift_right_logical(pos, shift)
            vals = x_v[pl.ds(j * _L, _L)].astype(jnp.int32)
            idx_v[pl.ds(j * _L, _L)] = col * vocab + vals
            return 0

        lax.fori_loop(0, n_steps, body, 0, unroll=8)

        pltpu.async_copy(table_hbm.at[idx_v], out_v, sem).wait()
        pltpu.sync_copy(out_v, out_hbm.at[pl.ds(base, per_w)])

    return gather_kernel


def kernel(x, emb_tables):
    b, f = x.shape
    vocab = emb_tables.shape[1]
    table_flat = jnp.concatenate([emb_tables[i, :, 0] for i in range(f)])
    xc = x.T.reshape(-1)
    out_cm = _build(b, f, vocab)(table_flat, xc)
    return out_cm.reshape(f, b).T


# speedup vs baseline: 2.7231x; 2.7231x over previous
import functools

import jax
import jax.numpy as jnp
from jax import lax
from jax.experimental import pallas as pl
from jax.experimental.pallas import tpu as pltpu
from jax.experimental.pallas import tpu_sc as plsc

_L = 16


@functools.cache
def _build(batch: int, n_fields: int, vocab: int):
    info = plsc.get_sparse_core_info()
    nc, ns = info.num_cores, info.num_subcores
    nw = nc * ns
    assert batch % (nw * _L) == 0
    b_w = batch // nw
    per_w = b_w * n_fields

    mesh = plsc.VectorSubcoreMesh(
        core_axis_name="c", subcore_axis_name="s", num_cores=nc,
        num_subcores=ns)

    @functools.partial(
        pl.kernel,
        out_type=jax.ShapeDtypeStruct((n_fields * batch,), jnp.float32),
        mesh=mesh,
        scratch_types=[
            pltpu.VMEM((per_w,), jnp.float32),
            pltpu.VMEM((per_w,), jnp.int32),
            pltpu.VMEM((per_w,), jnp.float32),
            pltpu.SemaphoreType.DMA,
        ],
    )
    def gather_kernel(*refs):
        tables = refs[:n_fields]
        xc_hbm, out_hbm, x_v, idx_v, g_v, sem = refs[n_fields:]
        wid = lax.axis_index("s") * nc + lax.axis_index("c")
        b0 = wid * b_w

        for i in range(n_fields):
            pltpu.sync_copy(xc_hbm.at[pl.ds(i * batch + b0, b_w)],
                            x_v.at[pl.ds(i * b_w, b_w)])

        def body(j, _):
            k = pl.ds(j * _L, _L)
            idx_v[k] = x_v[k].astype(jnp.int32)
            return 0

        lax.fori_loop(0, per_w // _L, body, 0, unroll=4)

        copies = []
        for i in range(n_fields):
            sl = pl.ds(i * b_w, b_w)
            copies.append(
                pltpu.async_copy(tables[i].at[idx_v.at[sl]], g_v.at[sl],
                                 sem))
        for c in copies:
            c.wait()
        for i in range(n_fields):
            sl = pl.ds(i * b_w, b_w)
            pltpu.sync_copy(g_v.at[sl],
                            out_hbm.at[pl.ds(i * batch + b0, b_w)])

    return gather_kernel


def kernel(x, emb_tables):
    b, f = x.shape
    gather_kernel = _build(b, f, emb_tables.shape[1])
    cols = [emb_tables[i, :, 0] for i in range(f)]
    xc = x.T.reshape(-1)
    out_cm = gather_kernel(*cols, xc)
    return out_cm.reshape(f, b).T


# speedup vs baseline: 2.8303x; 1.0394x over previous
import functools

import jax
import jax.numpy as jnp
from jax import lax
from jax.experimental import pallas as pl
from jax.experimental.pallas import tpu as pltpu
from jax.experimental.pallas import tpu_sc as plsc

_L = 16


@functools.cache
def _build(batch: int, n_fields: int, total_fields: int, col0: int,
           vocab: int):
    info = plsc.get_sparse_core_info()
    nc, ns = info.num_cores, info.num_subcores
    nw = nc * ns
    assert batch % (nw * _L) == 0
    b_w = batch // nw
    per_w = b_w * n_fields

    mesh = plsc.VectorSubcoreMesh(
        core_axis_name="c", subcore_axis_name="s", num_cores=nc,
        num_subcores=ns)

    @functools.partial(
        pl.kernel,
        out_type=jax.ShapeDtypeStruct((n_fields * batch,), jnp.float32),
        mesh=mesh,
        scratch_types=[
            pltpu.VMEM((per_w,), jnp.float32),
            pltpu.VMEM((per_w,), jnp.int32),
            pltpu.VMEM((per_w,), jnp.float32),
            pltpu.SemaphoreType.DMA,
        ],
    )
    def gather_kernel(*refs):
        tables = refs[:n_fields]
        xc_hbm, out_hbm, x_v, idx_v, g_v, sem = refs[n_fields:]
        wid = lax.axis_index("s") * nc + lax.axis_index("c")
        b0 = wid * b_w

        for i in range(n_fields):
            pltpu.sync_copy(
                xc_hbm.at[pl.ds((col0 + i) * batch + b0, b_w)],
                x_v.at[pl.ds(i * b_w, b_w)])

        def body(j, _):
            k = pl.ds(j * _L, _L)
            idx_v[k] = x_v[k].astype(jnp.int32)
            return 0

        lax.fori_loop(0, per_w // _L, body, 0, unroll=4)

        copies = []
        for i in range(n_fields):
            sl = pl.ds(i * b_w, b_w)
            copies.append(
                pltpu.async_copy(tables[i].at[idx_v.at[sl]], g_v.at[sl],
                                 sem))
        for c in copies:
            c.wait()
        for i in range(n_fields):
            sl = pl.ds(i * b_w, b_w)
            pltpu.sync_copy(g_v.at[sl],
                            out_hbm.at[pl.ds(i * batch + b0, b_w)])

    return gather_kernel


def kernel(x, emb_tables):
    b, f = x.shape
    vocab = emb_tables.shape[1]
    cols = [emb_tables[i, :, 0] for i in range(f)]
    xc = x.T.reshape(-1)
    f0 = f // 2
    k0 = _build(b, f0, f, 0, vocab)
    k1 = _build(b, f - f0, f, f0, vocab)
    out0 = k0(*cols[:f0], xc)
    out1 = k1(*cols[f0:], xc)
    out_cm = jnp.concatenate([out0, out1])
    return out_cm.reshape(f, b).T
